# Initial kernel scaffold; baseline (speedup 1.0000x reference)
import jax
import jax.numpy as jnp
from jax.experimental import pallas as pl


def kernel(text, text_len, year, table, W1, b1, W2, b2, W3, b3):
    raise NotImplementedError("write your pallas kernel here")


# speedup vs baseline: 9.3815x; 9.3815x over previous
import functools

import jax
import jax.numpy as jnp
from jax import lax
from jax.experimental import pallas as pl
from jax.experimental.pallas import tpu as pltpu
from jax.experimental.pallas import tpu_sc as plsc

B, L, V, D, H, C = 16384, 50, 100000, 128, 50, 10

NC, NS = 2, 16
NW = NC * NS
ROWS_PER_W = B // NW
CHUNK_ROWS = 8
CHUNK_IDX = CHUNK_ROWS * L
NCHUNK = ROWS_PER_W // CHUNK_ROWS
GSPLIT = 5
GSIZE = CHUNK_IDX // GSPLIT
NLANE = 16
NVREG = D // NLANE


def _sc_embed_body(text_hbm, table_hbm, out_hbm, idx_v, rows_v, obuf, sem):
    wid = lax.axis_index("s") * NC + lax.axis_index("c")
    base_idx = wid * (ROWS_PER_W * L)
    base_row = wid * ROWS_PER_W

    def chunk_body(c, _):
        pltpu.sync_copy(text_hbm.at[pl.ds(base_idx + c * CHUNK_IDX, CHUNK_IDX)],
                        idx_v)
        cps = []
        for j in range(GSPLIT):
            cps.append(pltpu.async_copy(
                table_hbm.at[idx_v.at[pl.ds(j * GSIZE, GSIZE)]],
                rows_v.at[pl.ds(j * GSIZE, GSIZE)],
                sem))
        for cp in cps:
            cp.wait()
        for r in range(CHUNK_ROWS):
            def lbody(l, accs):
                row = r * L + l
                return tuple(accs[d] + rows_v[row, pl.ds(NLANE * d, NLANE)]
                             for d in range(NVREG))
            accs = lax.fori_loop(
                0, L, lbody,
                tuple(jnp.zeros((NLANE,), jnp.float32) for _ in range(NVREG)))
            for d in range(NVREG):
                obuf[r, pl.ds(NLANE * d, NLANE)] = accs[d] * (1.0 / L)
        pltpu.sync_copy(obuf,
                        out_hbm.at[pl.ds(base_row + c * CHUNK_ROWS,
                                         CHUNK_ROWS)])
        return 0

    lax.fori_loop(0, NCHUNK, chunk_body, 0)


_sc_embed = pl.kernel(
    _sc_embed_body,
    out_type=jax.ShapeDtypeStruct((B, D), jnp.float32),
    mesh=plsc.VectorSubcoreMesh(core_axis_name="c", subcore_axis_name="s",
                                num_cores=NC, num_subcores=NS),
    scratch_types=[
        pltpu.VMEM((CHUNK_IDX,), jnp.int32),
        pltpu.VMEM((CHUNK_IDX, D), jnp.float32),
        pltpu.VMEM((CHUNK_ROWS, D), jnp.float32),
        pltpu.SemaphoreType.DMA,
    ],
)


def _mlp_body(pooled_ref, ly_ref, W1a_ref, W1b_ref, b1_ref, W2_ref, b2_ref,
              W3_ref, b3_ref, out_ref):
    x = pooled_ref[...]
    h = jnp.dot(x, W1a_ref[...], preferred_element_type=jnp.float32)
    h += jnp.dot(ly_ref[...], W1b_ref[...], preferred_element_type=jnp.float32)
    h = jnp.maximum(h + b1_ref[...], 0.0)
    h = jnp.maximum(
        jnp.dot(h, W2_ref[...], preferred_element_type=jnp.float32)
        + b2_ref[...], 0.0)
    out_ref[...] = (jnp.dot(h, W3_ref[...], preferred_element_type=jnp.float32)
                    + b3_ref[...])


BM = 2048


def _mlp(pooled, ly, W1a, W1b, b1, W2, b2, W3, b3):
    grid = (B // BM,)
    return pl.pallas_call(
        _mlp_body,
        grid=grid,
        in_specs=[
            pl.BlockSpec((BM, D), lambda i: (i, 0)),
            pl.BlockSpec((BM, 2), lambda i: (i, 0)),
            pl.BlockSpec((D, H), lambda i: (0, 0)),
            pl.BlockSpec((2, H), lambda i: (0, 0)),
            pl.BlockSpec((1, H), lambda i: (0, 0)),
            pl.BlockSpec((H, H), lambda i: (0, 0)),
            pl.BlockSpec((1, H), lambda i: (0, 0)),
            pl.BlockSpec((H, C), lambda i: (0, 0)),
            pl.BlockSpec((1, C), lambda i: (0, 0)),
        ],
        out_specs=pl.BlockSpec((BM, C), lambda i: (i, 0)),
        out_shape=jax.ShapeDtypeStruct((B, C), jnp.float32),
    )(pooled, ly, W1a, W1b, b1, W2, b2, W3, b3)


@jax.jit
def kernel(text, text_len, year, table, W1, b1, W2, b2, W3, b3):
    text_flat = text.astype(jnp.int32).reshape(-1)
    pooled = _sc_embed(text_flat, table)
    ly = jnp.stack([text_len.astype(jnp.float32),
                    year.astype(jnp.float32)], axis=1)
    W1a = W1[:D]
    W1b = W1[D:]
    return _mlp(pooled, ly, W1a, W1b, b1.reshape(1, H), W2, b2.reshape(1, H),
                W3, b3.reshape(1, C))


# speedup vs baseline: 15.1932x; 1.6195x over previous
import functools

import jax
import jax.numpy as jnp
from jax import lax
from jax.experimental import pallas as pl
from jax.experimental.pallas import tpu as pltpu
from jax.experimental.pallas import tpu_sc as plsc

B, L, V, D, H, C = 16384, 50, 100000, 128, 50, 10

NC, NS = 2, 16
NW = NC * NS
ROWS_PER_W = B // NW
CHUNK_ROWS = 8
CHUNK_IDX = CHUNK_ROWS * L
NCHUNK = ROWS_PER_W // CHUNK_ROWS
GSPLIT = 5
GSIZE = CHUNK_IDX // GSPLIT
NLANE = 16
NVREG = D // NLANE


def _sc_embed_body(text_hbm, table_hbm, out_hbm, idx_v0, idx_v1, rows_v0,
                   rows_v1, obuf, sem0, sem1):
    wid = lax.axis_index("s") * NC + lax.axis_index("c")
    base_idx = wid * (ROWS_PER_W * L)
    base_row = wid * ROWS_PER_W
    sems = (sem0, sem1)
    idxs = (idx_v0, idx_v1)
    rows = (rows_v0, rows_v1)

    def fire(b, c):
        pltpu.sync_copy(text_hbm.at[pl.ds(base_idx + c * CHUNK_IDX, CHUNK_IDX)],
                        idxs[b])
        for j in range(GSPLIT):
            pltpu.async_copy(
                table_hbm.at[idxs[b].at[pl.ds(j * GSIZE, GSIZE)]],
                rows[b].at[pl.ds(j * GSIZE, GSIZE)],
                sems[b])

    def drain(b):
        for j in range(GSPLIT):
            pltpu.make_async_copy(
                table_hbm.at[idxs[b].at[pl.ds(j * GSIZE, GSIZE)]],
                rows[b].at[pl.ds(j * GSIZE, GSIZE)],
                sems[b]).wait()

    def reduce(b, c):
        for r in range(CHUNK_ROWS):
            def lbody(l, accs):
                row = r * L + l
                return tuple(accs[d] + rows[b][row, pl.ds(NLANE * d, NLANE)]
                             for d in range(NVREG))
            accs = lax.fori_loop(
                0, L, lbody,
                tuple(jnp.zeros((NLANE,), jnp.float32)
                      for _ in range(NVREG)))
            for d in range(NVREG):
                obuf[r, pl.ds(NLANE * d, NLANE)] = accs[d] * (1.0 / L)
        pltpu.sync_copy(obuf,
                        out_hbm.at[pl.ds(base_row + c * CHUNK_ROWS,
                                         CHUNK_ROWS)])

    fire(0, 0)

    def chunk_body(i, _):
        c0 = 2 * i
        fire(1, c0 + 1)
        drain(0)
        reduce(0, c0)
        fire(0, jnp.minimum(c0 + 2, NCHUNK - 1))
        drain(1)
        reduce(1, c0 + 1)
        return 0

    lax.fori_loop(0, NCHUNK // 2, chunk_body, 0)
    drain(0)


_sc_embed = pl.kernel(
    _sc_embed_body,
    out_type=jax.ShapeDtypeStruct((B, D), jnp.float32),
    mesh=plsc.VectorSubcoreMesh(core_axis_name="c", subcore_axis_name="s",
                                num_cores=NC, num_subcores=NS),
    scratch_types=[
        pltpu.VMEM((CHUNK_IDX,), jnp.int32),
        pltpu.VMEM((CHUNK_IDX,), jnp.int32),
        pltpu.VMEM((CHUNK_IDX, D), jnp.float32),
        pltpu.VMEM((CHUNK_IDX, D), jnp.float32),
        pltpu.VMEM((CHUNK_ROWS, D), jnp.float32),
        pltpu.SemaphoreType.DMA,
        pltpu.SemaphoreType.DMA,
    ],
)


def _mlp_body(pooled_ref, ly_ref, W1a_ref, W1b_ref, b1_ref, W2_ref, b2_ref,
              W3_ref, b3_ref, out_ref):
    x = pooled_ref[...]
    h = jnp.dot(x, W1a_ref[...], preferred_element_type=jnp.float32)
    h += jnp.dot(ly_ref[...], W1b_ref[...], preferred_element_type=jnp.float32)
    h = jnp.maximum(h + b1_ref[...], 0.0)
    h = jnp.maximum(
        jnp.dot(h, W2_ref[...], preferred_element_type=jnp.float32)
        + b2_ref[...], 0.0)
    out_ref[...] = (jnp.dot(h, W3_ref[...], preferred_element_type=jnp.float32)
                    + b3_ref[...])


BM = 2048


def _mlp(pooled, ly, W1a, W1b, b1, W2, b2, W3, b3):
    grid = (B // BM,)
    return pl.pallas_call(
        _mlp_body,
        grid=grid,
        in_specs=[
            pl.BlockSpec((BM, D), lambda i: (i, 0)),
            pl.BlockSpec((BM, 2), lambda i: (i, 0)),
            pl.BlockSpec((D, H), lambda i: (0, 0)),
            pl.BlockSpec((2, H), lambda i: (0, 0)),
            pl.BlockSpec((1, H), lambda i: (0, 0)),
            pl.BlockSpec((H, H), lambda i: (0, 0)),
            pl.BlockSpec((1, H), lambda i: (0, 0)),
            pl.BlockSpec((H, C), lambda i: (0, 0)),
            pl.BlockSpec((1, C), lambda i: (0, 0)),
        ],
        out_specs=pl.BlockSpec((BM, C), lambda i: (i, 0)),
        out_shape=jax.ShapeDtypeStruct((B, C), jnp.float32),
    )(pooled, ly, W1a, W1b, b1, W2, b2, W3, b3)


@jax.jit
def kernel(text, text_len, year, table, W1, b1, W2, b2, W3, b3):
    text_flat = text.astype(jnp.int32).reshape(-1)
    pooled = _sc_embed(text_flat, table)
    ly = jnp.stack([text_len.astype(jnp.float32),
                    year.astype(jnp.float32)], axis=1)
    W1a = W1[:D]
    W1b = W1[D:]
    return _mlp(pooled, ly, W1a, W1b, b1.reshape(1, H), W2, b2.reshape(1, H),
                W3, b3.reshape(1, C))
